# Initial kernel scaffold; baseline (speedup 1.0000x reference)
import jax
import jax.numpy as jnp
from jax.experimental import pallas as pl


def kernel(x, edge_index, edge_pairs, W1, b1, W2, b2, W3, b3, Wc, bc):
    raise NotImplementedError("write your pallas kernel here")


# speedup vs baseline: 10.7711x; 10.7711x over previous
import functools
import jax
import jax.numpy as jnp
from jax import lax
from jax.experimental import pallas as pl
from jax.experimental.pallas import tpu as pltpu
from jax.experimental.pallas import tpu_sc as plsc

N = 10000
E = 160000
IN_DIM = 768
H = 128
NC, NS = 2, 16
CHUNK = 128
D = H // NC
D3 = 16
ROWS = 1024
GRID = 10
NPAD = ROWS * GRID
EPAD = NC * NS * 40 * CHUNK
EW = EPAD // NS
EW2 = EPAD // (NC * NS)
RP = NPAD // NS


def _mesh():
    return plsc.VectorSubcoreMesh(
        core_axis_name="c", subcore_axis_name="s", num_cores=NC, num_subcores=NS
    )


def _deg_call(dstp):
    @functools.partial(
        pl.kernel,
        out_type=jax.ShapeDtypeStruct((NC, NPAD, D3), jnp.float32),
        mesh=_mesh(),
        compiler_params=pltpu.CompilerParams(use_tc_tiling_on_sc=False),
        scratch_types=[
            pltpu.VMEM((CHUNK, D3), jnp.float32),
            pltpu.VMEM((EW2 // CHUNK, 1, CHUNK), jnp.int32),
            pltpu.VMEM_SHARED((NPAD, D3), jnp.float32),
        ],
    )
    def k(dst_hbm, out_hbm, buf_v, di_all, acc):
        c = lax.axis_index("c")
        s = lax.axis_index("s")
        w = c * NS + s

        @pl.loop(0, CHUNK)
        def _(i):
            buf_v[i, :] = jnp.zeros((D3,), jnp.float32)

        @pl.loop(0, RP // CHUNK)
        def _(j):
            pltpu.sync_copy(buf_v, acc.at[pl.ds(s * RP + j * CHUNK, CHUNK)])

        @pl.loop(0, CHUNK)
        def _(i):
            buf_v[i, :] = jnp.ones((D3,), jnp.float32)

        pltpu.sync_copy(dst_hbm.at[w], di_all)
        plsc.subcore_barrier()

        @pl.loop(0, EW2 // CHUNK)
        def _(j):
            pltpu.sync_copy(buf_v, acc.at[di_all.at[j, 0]], add=True)

        plsc.subcore_barrier()
        pltpu.sync_copy(acc.at[pl.ds(s * RP, RP)], out_hbm.at[c, pl.ds(s * RP, RP)])

    return k(dstp)


def _conv_call(gsplit, srcp, dstp):
    @functools.partial(
        pl.kernel,
        out_type=jax.ShapeDtypeStruct((NC, NPAD, D), jnp.float32),
        mesh=_mesh(),
        compiler_params=pltpu.CompilerParams(use_tc_tiling_on_sc=False),
        scratch_types=[
            pltpu.VMEM((EW,), jnp.int32),
            pltpu.VMEM((EW // CHUNK, 1, CHUNK), jnp.int32),
            pltpu.VMEM((CHUNK, D), jnp.float32),
            pltpu.VMEM_SHARED((NPAD, D), jnp.float32),
        ],
    )
    def k(g_hbm, src_hbm, dst_hbm, out_hbm, si_all, di_all, rows_v, acc):
        c = lax.axis_index("c")
        s = lax.axis_index("s")

        pltpu.sync_copy(src_hbm.at[pl.ds(s * EW, EW)], si_all)
        pltpu.sync_copy(dst_hbm.at[s], di_all)
        pltpu.sync_copy(g_hbm.at[c, pl.ds(s * RP, RP)], acc.at[pl.ds(s * RP, RP)])
        plsc.subcore_barrier()

        @pl.loop(0, EW // CHUNK)
        def _(j):
            pltpu.sync_copy(g_hbm.at[c].at[si_all.at[pl.ds(j * CHUNK, CHUNK)]], rows_v)
            pltpu.sync_copy(rows_v, acc.at[di_all.at[j, 0]], add=True)

        plsc.subcore_barrier()
        pltpu.sync_copy(acc.at[pl.ds(s * RP, RP)], out_hbm.at[c, pl.ds(s * RP, RP)])

    return k(gsplit, srcp, dstp)


def _conv3_call(g3, srcp, dstp):
    @functools.partial(
        pl.kernel,
        out_type=jax.ShapeDtypeStruct((NC, NPAD, D3), jnp.float32),
        mesh=_mesh(),
        compiler_params=pltpu.CompilerParams(use_tc_tiling_on_sc=False),
        scratch_types=[
            pltpu.VMEM((EW2,), jnp.int32),
            pltpu.VMEM((EW2 // CHUNK, 1, CHUNK), jnp.int32),
            pltpu.VMEM((CHUNK, D3), jnp.float32),
            pltpu.VMEM_SHARED((NPAD, D3), jnp.float32),
        ],
    )
    def k(g_hbm, src_hbm, dst_hbm, out_hbm, si_all, di_all, rows_v, acc):
        c = lax.axis_index("c")
        s = lax.axis_index("s")
        w = c * NS + s

        pltpu.sync_copy(src_hbm.at[pl.ds(w * EW2, EW2)], si_all)
        pltpu.sync_copy(dst_hbm.at[w], di_all)
        pltpu.sync_copy(g_hbm.at[pl.ds(s * RP, RP)], acc.at[pl.ds(s * RP, RP)])
        plsc.subcore_barrier()

        @pl.loop(0, EW2 // CHUNK)
        def _(j):
            pltpu.sync_copy(g_hbm.at[si_all.at[pl.ds(j * CHUNK, CHUNK)]], rows_v)
            pltpu.sync_copy(rows_v, acc.at[di_all.at[j, 0]], add=True)

        plsc.subcore_barrier()
        pltpu.sync_copy(acc.at[pl.ds(s * RP, RP)], out_hbm.at[c, pl.ds(s * RP, RP)])

    return k(g3, srcp, dstp)


def _pair_call(u, v, p0p, p1p):
    @functools.partial(
        pl.kernel,
        out_type=jax.ShapeDtypeStruct((EPAD,), jnp.float32),
        mesh=_mesh(),
        compiler_params=pltpu.CompilerParams(use_tc_tiling_on_sc=False, needs_layout_passes=False),
        scratch_types=[
            pltpu.VMEM((NPAD,), jnp.float32),
            pltpu.VMEM((NPAD,), jnp.float32),
            pltpu.VMEM((EW2,), jnp.int32),
            pltpu.VMEM((EW2,), jnp.int32),
            pltpu.VMEM((EW2,), jnp.float32),
        ],
    )
    def k(u_hbm, v_hbm, p0_hbm, p1_hbm, out_hbm, u_v, v_v, i0_v, i1_v, o_v):
        c = lax.axis_index("c")
        s = lax.axis_index("s")
        w = c * NS + s
        base = w * EW2

        pltpu.sync_copy(u_hbm, u_v)
        pltpu.sync_copy(v_hbm, v_v)
        pltpu.sync_copy(p0_hbm.at[pl.ds(base, EW2)], i0_v)
        pltpu.sync_copy(p1_hbm.at[pl.ds(base, EW2)], i1_v)

        @pl.loop(0, EW2 // 16)
        def _(j):
            i0 = i0_v[pl.ds(j * 16, 16)]
            i1 = i1_v[pl.ds(j * 16, 16)]
            a = plsc.load_gather(u_v, [i0])
            b = plsc.load_gather(v_v, [i1])
            o_v[pl.ds(j * 16, 16)] = a + b

        pltpu.sync_copy(o_v, out_hbm.at[pl.ds(base, EW2)])

    return k(u, v, p0p, p1p)


def _mm_call(x, W1):
    def body(x_ref, w_ref, o_ref):
        o_ref[...] = jnp.dot(x_ref[...], w_ref[...],
                             preferred_element_type=jnp.float32)

    return pl.pallas_call(
        body,
        grid=(GRID,),
        in_specs=[
            pl.BlockSpec((ROWS, IN_DIM), lambda i: (i, 0)),
            pl.BlockSpec((IN_DIM, H), lambda i: (0, 0)),
        ],
        out_specs=pl.BlockSpec((ROWS, H), lambda i: (i, 0)),
        out_shape=jax.ShapeDtypeStruct((NPAD, H), jnp.float32),
    )(x, W1)


def _scale_split_call(h, degp):
    def body(h_ref, deg_ref, g_ref, dinv_ref):
        deg = deg_ref[0, :, 0:1] + deg_ref[1, :, 0:1] + 1.0
        dinv = lax.rsqrt(deg)
        dinv_ref[...] = dinv
        g = h_ref[...] * dinv
        g_ref[0] = g[:, :D]
        g_ref[1] = g[:, D:]

    return pl.pallas_call(
        body,
        grid=(GRID,),
        in_specs=[
            pl.BlockSpec((ROWS, H), lambda i: (i, 0)),
            pl.BlockSpec((NC, ROWS, D3), lambda i: (0, i, 0)),
        ],
        out_specs=[
            pl.BlockSpec((NC, ROWS, D), lambda i: (0, i, 0)),
            pl.BlockSpec((ROWS, 1), lambda i: (i, 0)),
        ],
        out_shape=[
            jax.ShapeDtypeStruct((NC, NPAD, D), jnp.float32),
            jax.ShapeDtypeStruct((NPAD, 1), jnp.float32),
        ],
    )(h, degp)


def _mid_call(y, dinv, b, W, Wuv):
    def body(y_ref, dinv_ref, b_ref, w_ref, wuv_ref, g_ref, uv_ref):
        yb = jnp.concatenate([y_ref[0], y_ref[1]], axis=1)
        dinv = dinv_ref[...]
        xb = jnp.tanh(dinv * yb + b_ref[...])
        g = jnp.dot(xb, w_ref[...], preferred_element_type=jnp.float32) * dinv
        g_ref[0] = g[:, :D]
        g_ref[1] = g[:, D:]
        uv_ref[...] = jnp.dot(xb, wuv_ref[...], preferred_element_type=jnp.float32)

    return pl.pallas_call(
        body,
        grid=(GRID,),
        in_specs=[
            pl.BlockSpec((NC, ROWS, D), lambda i: (0, i, 0)),
            pl.BlockSpec((ROWS, 1), lambda i: (i, 0)),
            pl.BlockSpec((1, H), lambda i: (0, 0)),
            pl.BlockSpec((H, H), lambda i: (0, 0)),
            pl.BlockSpec((H, 2), lambda i: (0, 0)),
        ],
        out_specs=[
            pl.BlockSpec((NC, ROWS, D), lambda i: (0, i, 0)),
            pl.BlockSpec((ROWS, 2), lambda i: (i, 0)),
        ],
        out_shape=[
            jax.ShapeDtypeStruct((NC, NPAD, D), jnp.float32),
            jax.ShapeDtypeStruct((NPAD, 2), jnp.float32),
        ],
    )(y, dinv, b, W, Wuv)


def _lay3_call(y, dinv, b, W3, Wuv):
    def body(y_ref, dinv_ref, b_ref, w3_ref, wuv_ref, g3_ref, uv_ref):
        yb = jnp.concatenate([y_ref[0], y_ref[1]], axis=1)
        dinv = dinv_ref[...]
        xb = jnp.tanh(dinv * yb + b_ref[...])
        g3 = jnp.dot(xb, w3_ref[...], preferred_element_type=jnp.float32) * dinv
        g3_ref[...] = jnp.broadcast_to(g3, (ROWS, D3))
        uv_ref[...] = jnp.dot(xb, wuv_ref[...], preferred_element_type=jnp.float32)

    return pl.pallas_call(
        body,
        grid=(GRID,),
        in_specs=[
            pl.BlockSpec((NC, ROWS, D), lambda i: (0, i, 0)),
            pl.BlockSpec((ROWS, 1), lambda i: (i, 0)),
            pl.BlockSpec((1, H), lambda i: (0, 0)),
            pl.BlockSpec((H, 1), lambda i: (0, 0)),
            pl.BlockSpec((H, 2), lambda i: (0, 0)),
        ],
        out_specs=[
            pl.BlockSpec((ROWS, D3), lambda i: (i, 0)),
            pl.BlockSpec((ROWS, 2), lambda i: (i, 0)),
        ],
        out_shape=[
            jax.ShapeDtypeStruct((NPAD, D3), jnp.float32),
            jax.ShapeDtypeStruct((NPAD, 2), jnp.float32),
        ],
    )(y, dinv, b, W3, Wuv)


def _uv_call(y3p, g3, dinv, uv1, uv2, b3, wc3u, wc3v, bcv):
    def body(y3_ref, g3_ref, dinv_ref, uv1_ref, uv2_ref, b3_ref, wcu_ref,
             wcv_ref, bc_ref, u_ref, v_ref):
        y3 = y3_ref[0, :, 0:1] + y3_ref[1, :, 0:1] - g3_ref[:, 0:1]
        x3 = jnp.tanh(dinv_ref[...] * y3 + b3_ref[...])
        u_ref[...] = uv1_ref[:, 0:1] + uv2_ref[:, 0:1] + x3 * wcu_ref[...] + bc_ref[...]
        v_ref[...] = uv1_ref[:, 1:2] + uv2_ref[:, 1:2] + x3 * wcv_ref[...]

    one = pl.BlockSpec((1, 1), lambda i: (0, 0))
    return pl.pallas_call(
        body,
        grid=(GRID,),
        in_specs=[
            pl.BlockSpec((NC, ROWS, D3), lambda i: (0, i, 0)),
            pl.BlockSpec((ROWS, D3), lambda i: (i, 0)),
            pl.BlockSpec((ROWS, 1), lambda i: (i, 0)),
            pl.BlockSpec((ROWS, 2), lambda i: (i, 0)),
            pl.BlockSpec((ROWS, 2), lambda i: (i, 0)),
            one, one, one, one,
        ],
        out_specs=[
            pl.BlockSpec((ROWS, 1), lambda i: (i, 0)),
            pl.BlockSpec((ROWS, 1), lambda i: (i, 0)),
        ],
        out_shape=[
            jax.ShapeDtypeStruct((NPAD, 1), jnp.float32),
            jax.ShapeDtypeStruct((NPAD, 1), jnp.float32),
        ],
    )(y3p, g3, dinv, uv1, uv2, b3, wc3u, wc3v, bcv)


@jax.jit
def kernel(x, edge_index, edge_pairs, W1, b1, W2, b2, W3, b3, Wc, bc):
    f32 = jnp.float32
    src = edge_index[0].astype(jnp.int32)
    dst = edge_index[1].astype(jnp.int32)
    npad = jnp.full((EPAD - E,), N, jnp.int32)
    zpad = jnp.zeros((EPAD - E,), jnp.int32)
    srcp = jnp.concatenate([src, npad])
    dstp = jnp.concatenate([dst, npad])
    dst16 = dstp.reshape(NS, EW // CHUNK, 1, CHUNK)
    dst32 = dstp.reshape(NC * NS, EW2 // CHUNK, 1, CHUNK)
    p0p = jnp.concatenate([edge_pairs[:, 0].astype(jnp.int32), zpad])
    p1p = jnp.concatenate([edge_pairs[:, 1].astype(jnp.int32), zpad])

    wc = Wc[:, 0]
    Wuv1 = jnp.stack([wc[0:H], wc[257:257 + H]], axis=1)
    Wuv2 = jnp.stack([wc[H:2 * H], wc[257 + H:257 + 2 * H]], axis=1)
    wc3u = wc[256].reshape(1, 1)
    wc3v = wc[513].reshape(1, 1)
    bcv = bc.reshape(1, 1).astype(f32)

    degp = _deg_call(dst32)
    h1 = _mm_call(x, W1)
    gs1, dinv = _scale_split_call(h1, degp)
    y1 = _conv_call(gs1, srcp, dst16)
    gs2, uv1 = _mid_call(y1, dinv, b1.reshape(1, H), W2, Wuv1)
    y2 = _conv_call(gs2, srcp, dst16)
    g3, uv2 = _lay3_call(y2, dinv, b2.reshape(1, H), W3, Wuv2)
    y3p = _conv3_call(g3, srcp, dst32)
    u, v = _uv_call(y3p, g3, dinv, uv1, uv2, b3.reshape(1, 1), wc3u, wc3v, bcv)
    o = _pair_call(u.reshape(NPAD), v.reshape(NPAD), p0p, p1p)
    return o[:E]
